# baseline (speedup 1.0000x reference)
import jax
import jax.numpy as jnp
from jax.experimental import pallas as pl


def _gate_body(z_ref, tf_ref, tr_ref, w1a_ref, wtail_ref, b1_ref, a_ref,
               lnw_ref, lnb_ref, w2_ref, b2_ref, out_ref):
    x = z_ref[...]
    prod = w1a_ref[...] * x
    h = jnp.sum(prod, axis=1, keepdims=True)
    h = h + wtail_ref[:, 0:1] * tf_ref[...] + wtail_ref[:, 1:2] * tr_ref[...]
    h = h + b1_ref[...]
    h = jnp.maximum(h, 0.0) + a_ref[...] * jnp.minimum(h, 0.0)
    mu = jnp.mean(h, axis=0, keepdims=True)
    d = h - mu
    var = jnp.mean(d * d, axis=0, keepdims=True)
    hn = d * jax.lax.rsqrt(var + 1e-5) * lnw_ref[...] + lnb_ref[...]
    logits = jax.lax.dot_general(
        w2_ref[...], hn, (((1,), (0,)), ((), ())),
        preferred_element_type=jnp.float32,
    ) + b2_ref[...]
    m = jnp.max(logits, axis=0, keepdims=True)
    e = jnp.exp(logits - m)
    s = jnp.sum(e, axis=0, keepdims=True)
    out_ref[...] = e / s


@jax.jit
def _gate(z, tf, tr, W1a, Wtail, b1, a, lnw, lnb, W2, b2):
    out = pl.pallas_call(
        _gate_body,
        out_shape=jax.ShapeDtypeStruct((64, 1), jnp.float32),
    )(z, tf, tr, W1a, Wtail, b1, a, lnw, lnb, W2, b2)
    return out.reshape(1, 64)


def kernel(z_refined, trust_form, trust_role, W1, b1, prelu_a, ln_w, ln_b, W2, b2):
    W1a = W1[:, :4096]
    Wtail = W1[:, 4096:]
    return _gate(
        z_refined,
        trust_form.reshape(1, 1),
        trust_role.reshape(1, 1),
        W1a,
        Wtail,
        b1.reshape(128, 1),
        prelu_a.reshape(1, 1),
        ln_w.reshape(128, 1),
        ln_b.reshape(128, 1),
        W2,
        b2.reshape(64, 1),
    )


# speedup vs baseline: 1.6370x; 1.6370x over previous
import jax
import jax.numpy as jnp
from jax.experimental import pallas as pl


def _dotT(a, b):
    return jax.lax.dot_general(
        a, b, (((1,), (1,)), ((), ())), preferred_element_type=jnp.float32
    )


def _gate_body(z_ref, tf_ref, tr_ref, w1_ref, b1_ref, a_ref,
               lnw_ref, lnb_ref, w2_ref, b2_ref, out_ref):
    h = _dotT(z_ref[...], w1_ref[:, 0:4096])
    h = h + _dotT(tf_ref[...], w1_ref[:, 4096:4097])
    h = h + _dotT(tr_ref[...], w1_ref[:, 4097:4098])
    h = h + b1_ref[...]
    h = jnp.maximum(h, 0.0) + a_ref[...] * jnp.minimum(h, 0.0)
    mu = jnp.mean(h, axis=1, keepdims=True)
    d = h - mu
    var = jnp.mean(d * d, axis=1, keepdims=True)
    hn = d * jax.lax.rsqrt(var + 1e-5) * lnw_ref[...] + lnb_ref[...]
    logits = _dotT(hn, w2_ref[...]) + b2_ref[...]
    m = jnp.max(logits, axis=1, keepdims=True)
    e = jnp.exp(logits - m)
    s = jnp.sum(e, axis=1, keepdims=True)
    out_ref[...] = e / s


@jax.jit
def _gate(z, tf, tr, W1, b1, a, lnw, lnb, W2, b2):
    return pl.pallas_call(
        _gate_body,
        out_shape=jax.ShapeDtypeStruct((1, 64), jnp.float32),
    )(z, tf, tr, W1, b1, a, lnw, lnb, W2, b2)


def kernel(z_refined, trust_form, trust_role, W1, b1, prelu_a, ln_w, ln_b, W2, b2):
    return _gate(
        z_refined,
        trust_form.reshape(1, 1),
        trust_role.reshape(1, 1),
        W1,
        b1.reshape(1, 128),
        prelu_a.reshape(1, 1),
        ln_w.reshape(1, 128),
        ln_b.reshape(1, 128),
        W2,
        b2.reshape(1, 64),
    )
